# baseline (speedup 1.0000x reference)
import functools

import jax
import jax.numpy as jnp
from jax import lax
from jax.experimental import pallas as pl
from jax.experimental.pallas import tpu as pltpu
from jax.experimental.pallas import tpu_sc as plsc

N_VOCAB = 1000000
N_EMBED = 64
BATCH = 16384
_HALF = N_VOCAB // 2

_info = plsc.get_sparse_core_info()
_NC, _NS = _info.num_cores, _info.num_subcores
_NW = _NC * _NS
_BPW = BATCH // _NW
_HB = 256

_mesh = plsc.VectorSubcoreMesh(core_axis_name="c", subcore_axis_name="s")


@functools.partial(
    pl.kernel,
    mesh=_mesh,
    compiler_params=pltpu.CompilerParams(
        use_tc_tiling_on_sc=False, needs_layout_passes=False),
    out_type=jax.ShapeDtypeStruct((BATCH, N_EMBED), jnp.float32),
    scratch_types=[
        pltpu.VMEM((_BPW,), jnp.int32),
        pltpu.VMEM((_BPW,), jnp.int32),
        pltpu.VMEM((_BPW,), jnp.int32),
        pltpu.VMEM((2 * _HB, N_EMBED), jnp.float32),
        pltpu.VMEM((_BPW, N_EMBED), jnp.float32),
        pltpu.SemaphoreType.DMA,
    ],
)
def _gather_kernel(idx_hbm, t0_hbm, t1_hbm, out_hbm, idx_v, ia_v, ib_v,
                   gbuf, rows_v, sem):
    wid = lax.axis_index("s") * _NC + lax.axis_index("c")
    base = wid * _BPW
    pltpu.sync_copy(idx_hbm.at[pl.ds(base, _BPW)], idx_v)

    def split_body(i, carry):
        vec = idx_v[pl.ds(i * 16, 16)]
        ia_v[pl.ds(i * 16, 16)] = jnp.minimum(vec, _HALF - 1)
        ib_v[pl.ds(i * 16, 16)] = jnp.maximum(vec - _HALF, 0)
        return carry

    lax.fori_loop(0, _BPW // 16, split_body, 0)

    col = lax.iota(jnp.int32, 16)

    for h in range(_BPW // _HB):
        copies = []
        for c in range(_HB // 128):
            o = h * _HB + c * 128
            copies.append(pltpu.async_copy(
                t0_hbm.at[ia_v.at[pl.ds(o, 128)]],
                gbuf.at[pl.ds(c * 128, 128)], sem))
            copies.append(pltpu.async_copy(
                t1_hbm.at[ib_v.at[pl.ds(o, 128)]],
                gbuf.at[pl.ds(_HB + c * 128, 128)], sem))
        for d in copies:
            d.wait()

        def extract_body(g, carry, h=h):
            vec = idx_v[pl.ds(h * _HB + g * 16, 16)]
            av = g * 16 + col + jnp.where(vec >= _HALF, _HB, 0)
            for j in range(16):
                row = h * _HB + g * 16 + j
                ri = jnp.full((16,), av[j], jnp.int32)
                wi = jnp.full((16,), row, jnp.int32)
                for k in range(N_EMBED // 16):
                    val = plsc.load_gather(gbuf, [ri, col + 16 * k])
                    plsc.store_scatter(rows_v, [wi, col + 16 * k], val)
            return carry

        lax.fori_loop(0, _HB // 16, extract_body, 0)

    pltpu.sync_copy(rows_v, out_hbm.at[pl.ds(base, _BPW)])


def kernel(input_words, in_embed):
    t0 = lax.slice(in_embed, (0, 0), (_HALF, N_EMBED))
    t1 = lax.slice(in_embed, (_HALF, 0), (N_VOCAB, N_EMBED))
    return _gather_kernel(input_words.astype(jnp.int32), t0, t1)


# speedup vs baseline: 1.7126x; 1.7126x over previous
import functools

import jax
import jax.numpy as jnp
from jax import lax
from jax.experimental import pallas as pl
from jax.experimental.pallas import tpu as pltpu
from jax.experimental.pallas import tpu_sc as plsc

N_VOCAB = 1000000
N_EMBED = 64
BATCH = 16384

_info = plsc.get_sparse_core_info()
_NC, _NS = _info.num_cores, _info.num_subcores
_NW = _NC * _NS
_BPW = BATCH // _NW
_CHUNK = 128
_NCHUNK = _BPW // _CHUNK

_mesh = plsc.VectorSubcoreMesh(core_axis_name="c", subcore_axis_name="s")


@functools.partial(
    pl.kernel,
    mesh=_mesh,
    compiler_params=pltpu.CompilerParams(
        use_tc_tiling_on_sc=False, skip_device_barrier=True),
    out_type=jax.ShapeDtypeStruct((BATCH, N_EMBED), jnp.float32),
    scratch_types=[
        pltpu.VMEM((_NCHUNK, _CHUNK), jnp.int32),
        pltpu.VMEM((_BPW, N_EMBED), jnp.float32),
        pltpu.SemaphoreType.DMA,
    ],
)
def _gather_kernel(idx_hbm, table_hbm, out_hbm, idx_v, rows_v, sem):
    wid = lax.axis_index("s") * _NC + lax.axis_index("c")
    base = wid * _BPW
    pltpu.sync_copy(idx_hbm.at[wid], idx_v)
    copies = []
    for j in range(_NCHUNK):
        copies.append(
            pltpu.async_copy(
                table_hbm.at[idx_v.at[j]],
                rows_v.at[pl.ds(j * _CHUNK, _CHUNK)],
                sem,
            )
        )
    for c in copies:
        c.wait()
    pltpu.sync_copy(rows_v, out_hbm.at[pl.ds(base, _BPW)])


def kernel(input_words, in_embed):
    idx = input_words.astype(jnp.int32).reshape(_NW, _NCHUNK, _CHUNK)
    return _gather_kernel(idx, in_embed)


# speedup vs baseline: 2.9582x; 1.7274x over previous
import functools

import jax
import jax.numpy as jnp
from jax import lax
from jax.experimental import pallas as pl
from jax.experimental.pallas import tpu as pltpu
from jax.experimental.pallas import tpu_sc as plsc

N_VOCAB = 1000000
N_EMBED = 64
BATCH = 16384

_info = plsc.get_sparse_core_info()
_NC, _NS = _info.num_cores, _info.num_subcores
_NW = _NC * _NS
_BPW = BATCH // _NW
_K = 16

_mesh = plsc.VectorSubcoreMesh(core_axis_name="c", subcore_axis_name="s")


@functools.partial(
    pl.kernel,
    mesh=_mesh,
    out_type=jax.ShapeDtypeStruct((BATCH, N_EMBED), jnp.float32),
    scratch_types=[
        pltpu.VMEM((_BPW,), jnp.int32),
        pltpu.VMEM((_BPW, N_EMBED), jnp.float32),
        pltpu.SemaphoreType.DMA,
    ],
)
def _gather_kernel(idx_hbm, table_hbm, out_hbm, idx_v, rows_v, sem):
    wid = lax.axis_index("s") * _NC + lax.axis_index("c")
    base = wid * _BPW
    pltpu.sync_copy(idx_hbm.at[pl.ds(base, _BPW)], idx_v)

    def issue_chunk(c, carry):
        cbase = c * _K
        vec = idx_v[pl.ds(cbase, _K)]
        for j in range(_K):
            w = vec[j]
            pltpu.async_copy(
                table_hbm.at[pl.ds(w, 1)],
                rows_v.at[pl.ds(cbase + j, 1)],
                sem,
            )
        return carry

    lax.fori_loop(0, _BPW // _K, issue_chunk, 0)
    pltpu.make_async_copy(table_hbm.at[pl.ds(0, _BPW)], rows_v, sem).wait()
    pltpu.sync_copy(rows_v, out_hbm.at[pl.ds(base, _BPW)])


def kernel(input_words, in_embed):
    return _gather_kernel(input_words.astype(jnp.int32), in_embed)
